# baseline (speedup 1.0000x reference)
import functools

import jax
import jax.numpy as jnp
from jax import lax
from jax.experimental import pallas as pl
from jax.experimental.pallas import tpu as pltpu
from jax.experimental.pallas import tpu_sc as plsc

D_IN = 4096
D_FEAT = 1024
BATCH = 4096

NC, NS, L = 2, 16, 16
NW = NC * NS
ROWS_PER_W = BATCH // NW
R = 8
CHUNKS = ROWS_PER_W // R
G = D_IN // L

_mesh = plsc.VectorSubcoreMesh(core_axis_name="c", subcore_axis_name="s")


@functools.partial(
    pl.kernel,
    out_type=jax.ShapeDtypeStruct((BATCH, D_FEAT), jnp.float32),
    mesh=_mesh,
    scratch_types=(
        [
            pltpu.VMEM((D_IN,), jnp.int32),
            pltpu.VMEM((D_IN,), jnp.float32),
            pltpu.VMEM((R, D_IN), jnp.float32),
            pltpu.VMEM((R, D_IN), jnp.float32),
        ]
        + [pltpu.VMEM((D_FEAT,), jnp.float32)] * (2 * R)
        + [pltpu.SemaphoreType.DMA] * 4
    ),
    compiler_params=pltpu.CompilerParams(needs_layout_passes=False),
)
def _count_sketch_sc(x_hbm, i_hbm, s_hbm, out_hbm,
                     ibuf, sbuf, xbuf0, xbuf1, *rest):
    accs = (rest[0:R], rest[R:2 * R])
    sem_in = (rest[2 * R], rest[2 * R + 1])
    sem_out = (rest[2 * R + 2], rest[2 * R + 3])
    xbufs = (xbuf0, xbuf1)

    wid = lax.axis_index("s") * NC + lax.axis_index("c")
    base = wid * ROWS_PER_W

    pltpu.sync_copy(i_hbm, ibuf)
    pltpu.sync_copy(s_hbm, sbuf)

    pltpu.async_copy(x_hbm.at[pl.ds(base, R)], xbuf0, sem_in[0])
    pltpu.async_copy(x_hbm.at[pl.ds(base + R, R)], xbuf1, sem_in[1])

    zv = jnp.zeros((L,), jnp.float32)

    @pl.loop(0, CHUNKS, step=2)
    def _chunk(c):
        for b in range(2):
            cc = c + b
            xb = xbufs[b]
            acc = accs[b]
            pltpu.make_async_copy(x_hbm.at[pl.ds(0, R)], xb, sem_in[b]).wait()

            @pl.when(c >= 2)
            def _drain():
                for r in range(R):
                    pltpu.make_async_copy(
                        out_hbm.at[0], acc[r], sem_out[b]).wait()

            @plsc.parallel_loop(0, D_FEAT // L, unroll=4)
            def _zero(k):
                for r in range(R):
                    acc[r][pl.ds(k * L, L)] = zv

            @plsc.parallel_loop(0, G, unroll=8)
            def _accum(g):
                iv = ibuf[pl.ds(g * L, L)]
                sv = sbuf[pl.ds(g * L, L)]
                for r in range(R):
                    xv = xb[r, pl.ds(g * L, L)]
                    plsc.addupdate_scatter(acc[r], [iv], xv * sv)

            @pl.when(cc + 2 < CHUNKS)
            def _refill():
                pltpu.async_copy(
                    x_hbm.at[pl.ds(base + (cc + 2) * R, R)], xb, sem_in[b])

            for r in range(R):
                pltpu.async_copy(
                    acc[r], out_hbm.at[base + cc * R + r], sem_out[b])

    for b in range(2):
        for r in range(R):
            pltpu.make_async_copy(
                out_hbm.at[0], accs[b][r], sem_out[b]).wait()


def kernel(x, i_hash, s_hash):
    return _count_sketch_sc(x, i_hash, s_hash)


# speedup vs baseline: 1.1677x; 1.1677x over previous
import functools

import jax
import jax.numpy as jnp
from jax import lax
from jax.experimental import pallas as pl
from jax.experimental.pallas import tpu as pltpu
from jax.experimental.pallas import tpu_sc as plsc

D_IN = 4096
D_FEAT = 1024
BATCH = 4096

NC, NS, L = 2, 16, 16
NW = NC * NS
R = 8
G = D_IN // L

SC_BATCH = 2560
ROWS_PER_W = SC_BATCH // NW
CHUNKS = ROWS_PER_W // R

_mesh = plsc.VectorSubcoreMesh(core_axis_name="c", subcore_axis_name="s")


@functools.partial(
    pl.kernel,
    out_type=jax.ShapeDtypeStruct((SC_BATCH, D_FEAT), jnp.float32),
    mesh=_mesh,
    scratch_types=(
        [
            pltpu.VMEM((D_IN,), jnp.int32),
            pltpu.VMEM((D_IN,), jnp.float32),
            pltpu.VMEM((R, D_IN), jnp.float32),
            pltpu.VMEM((R, D_IN), jnp.float32),
        ]
        + [pltpu.VMEM((D_FEAT,), jnp.float32)] * (2 * R)
        + [pltpu.SemaphoreType.DMA] * 4
    ),
    compiler_params=pltpu.CompilerParams(needs_layout_passes=False),
)
def _count_sketch_sc(x_hbm, i_hbm, s_hbm, out_hbm,
                     ibuf, sbuf, xbuf0, xbuf1, *rest):
    accs = (rest[0:R], rest[R:2 * R])
    sem_in = (rest[2 * R], rest[2 * R + 1])
    sem_out = (rest[2 * R + 2], rest[2 * R + 3])
    xbufs = (xbuf0, xbuf1)

    wid = lax.axis_index("s") * NC + lax.axis_index("c")
    base = wid * ROWS_PER_W

    pltpu.sync_copy(i_hbm, ibuf)
    pltpu.sync_copy(s_hbm, sbuf)

    pltpu.async_copy(x_hbm.at[pl.ds(base, R)], xbuf0, sem_in[0])
    pltpu.async_copy(x_hbm.at[pl.ds(base + R, R)], xbuf1, sem_in[1])

    zv = jnp.zeros((L,), jnp.float32)

    @pl.loop(0, CHUNKS, step=2)
    def _chunk(c):
        for b in range(2):
            cc = c + b
            xb = xbufs[b]
            acc = accs[b]
            pltpu.make_async_copy(x_hbm.at[pl.ds(0, R)], xb, sem_in[b]).wait()

            @pl.when(c >= 2)
            def _drain():
                for r in range(R):
                    pltpu.make_async_copy(
                        out_hbm.at[0], acc[r], sem_out[b]).wait()

            @plsc.parallel_loop(0, D_FEAT // L, unroll=4)
            def _zero(k):
                for r in range(R):
                    acc[r][pl.ds(k * L, L)] = zv

            @plsc.parallel_loop(0, G, unroll=4)
            def _accum(g):
                iv = ibuf[pl.ds(g * L, L)]
                sv = sbuf[pl.ds(g * L, L)]
                for r in range(R):
                    xv = xb[r, pl.ds(g * L, L)]
                    plsc.addupdate_scatter(acc[r], [iv], xv * sv)

            @pl.when(cc + 2 < CHUNKS)
            def _refill():
                pltpu.async_copy(
                    x_hbm.at[pl.ds(base + (cc + 2) * R, R)], xb, sem_in[b])

            for r in range(R):
                pltpu.async_copy(
                    acc[r], out_hbm.at[base + cc * R + r], sem_out[b])

    for b in range(2):
        for r in range(R):
            pltpu.make_async_copy(
                out_hbm.at[0], accs[b][r], sem_out[b]).wait()


TC_BATCH = BATCH - SC_BATCH
BM = 256
BK = 512
KT = D_IN // BK


def _tc_body(ih_ref, sh_ref, x_ref, o_ref):
    k = pl.program_id(1)
    ih = ih_ref[0, 0, :]
    sh = sh_ref[0, 0, :]
    cols = lax.broadcasted_iota(jnp.int32, (BK, D_FEAT), 1)
    onehot = jnp.where(ih[:, None] == cols, sh[:, None], 0.0).astype(jnp.bfloat16)
    part = jnp.dot(x_ref[...].astype(jnp.bfloat16), onehot,
                   preferred_element_type=jnp.float32)

    @pl.when(k == 0)
    def _init():
        o_ref[...] = part

    @pl.when(k > 0)
    def _acc():
        o_ref[...] += part


_tc_sketch = pl.pallas_call(
    _tc_body,
    grid=(TC_BATCH // BM, KT),
    in_specs=[
        pl.BlockSpec((1, 1, BK), lambda m, k: (k, 0, 0)),
        pl.BlockSpec((1, 1, BK), lambda m, k: (k, 0, 0)),
        pl.BlockSpec((BM, BK), lambda m, k: (m + SC_BATCH // BM, k)),
    ],
    out_specs=pl.BlockSpec((BM, D_FEAT), lambda m, k: (m, 0)),
    out_shape=jax.ShapeDtypeStruct((TC_BATCH, D_FEAT), jnp.float32),
    compiler_params=pltpu.CompilerParams(
        dimension_semantics=("parallel", "arbitrary")),
)


def kernel(x, i_hash, s_hash):
    out_sc = _count_sketch_sc(x, i_hash, s_hash)
    ih3 = i_hash.reshape(KT, 1, BK)
    sh3 = s_hash.reshape(KT, 1, BK)
    out_tc = _tc_sketch(ih3, sh3, x)
    return jnp.concatenate([out_sc, out_tc], axis=0)
